# baseline (speedup 1.0000x reference)
import functools

import jax
import jax.numpy as jnp
from jax import lax
from jax.experimental import pallas as pl
from jax.experimental.pallas import tpu as pltpu
from jax.experimental.pallas import tpu_sc as plsc

_MAX_LEN = 8192
_PE_DIM = 128
_D_MODEL = 1024
_BATCH = 4
_OUT_D = _D_MODEL + _PE_DIM

_NW = 32
_ROWS_PER_W = _BATCH * _MAX_LEN // _NW


@functools.partial(
    pl.kernel,
    mesh=plsc.VectorSubcoreMesh(core_axis_name="c", subcore_axis_name="s"),
    out_type=jax.ShapeDtypeStruct((_BATCH * _MAX_LEN, _OUT_D), jnp.float32),
    scratch_types=[pltpu.SemaphoreType.DMA, pltpu.SemaphoreType.DMA],
)
def _sc_concat(x_hbm, pe_hbm, out_hbm, sem_x, sem_pe):
    wid = lax.axis_index("s") * 2 + lax.axis_index("c")
    base = wid * _ROWS_PER_W
    t0 = base % _MAX_LEN
    cp_x = pltpu.make_async_copy(
        x_hbm.at[pl.ds(base, _ROWS_PER_W), :],
        out_hbm.at[pl.ds(base, _ROWS_PER_W), pl.ds(0, _D_MODEL)],
        sem_x,
    )
    cp_pe = pltpu.make_async_copy(
        pe_hbm.at[pl.ds(t0, _ROWS_PER_W), :],
        out_hbm.at[pl.ds(base, _ROWS_PER_W), pl.ds(_D_MODEL, _PE_DIM)],
        sem_pe,
    )
    cp_x.start()
    cp_pe.start()
    cp_x.wait()
    cp_pe.wait()


def kernel(x, pe_table):
    batch, max_len, d_model = x.shape
    x2 = x.reshape(batch * max_len, d_model)
    out = _sc_concat(x2, pe_table)
    return out.reshape(batch, max_len, _OUT_D)


# speedup vs baseline: 50.1624x; 50.1624x over previous
import jax
import jax.numpy as jnp
from jax.experimental import pallas as pl

_D_MODEL = 1024
_SEQ_BLK = 1024


def _body(x_ref, pe_ref, o_ref):
    o_ref[:, :, :_D_MODEL] = x_ref[...]
    o_ref[:, :, _D_MODEL:] = pe_ref[...][None]


def kernel(x, pe_table):
    batch, max_len, d_model = x.shape
    pe_dim = pe_table.shape[1]
    grid = (max_len // _SEQ_BLK, batch)
    return pl.pallas_call(
        _body,
        grid=grid,
        in_specs=[
            pl.BlockSpec((1, _SEQ_BLK, d_model), lambda s, b: (b, s, 0)),
            pl.BlockSpec((_SEQ_BLK, pe_dim), lambda s, b: (s, 0)),
        ],
        out_specs=pl.BlockSpec((1, _SEQ_BLK, d_model + pe_dim),
                               lambda s, b: (b, s, 0)),
        out_shape=jax.ShapeDtypeStruct((batch, max_len, d_model + pe_dim),
                                       x.dtype),
    )(x, pe_table)


# speedup vs baseline: 51.2494x; 1.0217x over previous
import jax
import jax.numpy as jnp
from jax.experimental import pallas as pl

_D_MODEL = 1024
_SEQ_BLK = 2048


def _body(x_ref, pe_ref, o_ref):
    o_ref[:, :, :_D_MODEL] = x_ref[...]
    o_ref[:, :, _D_MODEL:] = pe_ref[...][None]


def kernel(x, pe_table):
    batch, max_len, d_model = x.shape
    pe_dim = pe_table.shape[1]
    grid = (max_len // _SEQ_BLK, batch)
    return pl.pallas_call(
        _body,
        grid=grid,
        in_specs=[
            pl.BlockSpec((1, _SEQ_BLK, d_model), lambda s, b: (b, s, 0)),
            pl.BlockSpec((_SEQ_BLK, pe_dim), lambda s, b: (s, 0)),
        ],
        out_specs=pl.BlockSpec((1, _SEQ_BLK, d_model + pe_dim),
                               lambda s, b: (b, s, 0)),
        out_shape=jax.ShapeDtypeStruct((batch, max_len, d_model + pe_dim),
                                       x.dtype),
    )(x, pe_table)
